# baseline (speedup 1.0000x reference)
import functools

import jax
import jax.numpy as jnp
from jax import lax
from jax.experimental import pallas as pl
from jax.experimental.pallas import tpu as pltpu
from jax.experimental.pallas import tpu_sc as plsc

_NC = 2
_NS = 16
_NW = _NC * _NS
_CHUNK = 8
_NBUF = 12
_INFLIGHT = 8


@functools.cache
def _make_gather(B, D):
    b_per_w = B // _NW
    n_ch = b_per_w // _CHUNK
    mesh = plsc.VectorSubcoreMesh(core_axis_name="c", subcore_axis_name="s")

    @functools.partial(
        pl.kernel,
        mesh=mesh,
        out_type=jax.ShapeDtypeStruct((B, D), jnp.float32),
        scratch_types=(
            [pltpu.VMEM((n_ch, _CHUNK), jnp.int32)]
            + [pltpu.VMEM((_CHUNK, D), jnp.float32) for _ in range(_NBUF)]
            + [pltpu.SemaphoreType.DMA for _ in range(2 * _NBUF)]
        ),
    )
    def gather_kernel(idx_hbm, table_hbm, out_hbm, idx_v, *rest):
        bufs = rest[:_NBUF]
        gsems = rest[_NBUF:2 * _NBUF]
        ssems = rest[2 * _NBUF:3 * _NBUF]
        wid = lax.axis_index("s") * _NC + lax.axis_index("c")
        base = wid * b_per_w
        pltpu.sync_copy(idx_hbm.at[wid], idx_v)

        def gather(j):
            return pltpu.async_copy(
                table_hbm.at[idx_v.at[j]], bufs[j % _NBUF], gsems[j % _NBUF]
            )

        def scatter(j):
            return pltpu.async_copy(
                bufs[j % _NBUF],
                out_hbm.at[pl.ds(base + j * _CHUNK, _CHUNK)],
                ssems[j % _NBUF],
            )

        gathers = [None] * n_ch
        scatters = [None] * n_ch
        s_waited = [False] * n_ch
        prime = min(_INFLIGHT, n_ch)
        for j in range(prime):
            gathers[j] = gather(j)
        for j in range(n_ch):
            gathers[j].wait()
            scatters[j] = scatter(j)
            nx = j + prime
            if nx < n_ch:
                prev = nx - _NBUF
                if prev >= 0:
                    scatters[prev].wait()
                    s_waited[prev] = True
                gathers[nx] = gather(nx)
        for j in range(n_ch):
            if not s_waited[j]:
                scatters[j].wait()

    return gather_kernel


def kernel(x, W):
    Bx, S = x.shape
    D = W.shape[1]
    idx = x.reshape(-1).astype(jnp.int32)
    B = idx.size
    idx3 = idx.reshape(_NW, B // _NW // _CHUNK, _CHUNK)
    out = _make_gather(B, D)(idx3, W)
    return out.reshape(Bx, S, D)


# speedup vs baseline: 1.0266x; 1.0266x over previous
import functools

import jax
import jax.numpy as jnp
from jax import lax
from jax.experimental import pallas as pl
from jax.experimental.pallas import tpu as pltpu
from jax.experimental.pallas import tpu_sc as plsc

_NC = 2
_NS = 16
_NW = _NC * _NS
_CHUNK = 16
_NBUF = 7
_INFLIGHT = 6


@functools.cache
def _make_gather(B, D):
    b_per_w = B // _NW
    n_ch = b_per_w // _CHUNK
    mesh = plsc.VectorSubcoreMesh(core_axis_name="c", subcore_axis_name="s")

    @functools.partial(
        pl.kernel,
        mesh=mesh,
        out_type=jax.ShapeDtypeStruct((B, D), jnp.float32),
        scratch_types=(
            [pltpu.VMEM((n_ch, _CHUNK), jnp.int32)]
            + [pltpu.VMEM((_CHUNK, D), jnp.float32) for _ in range(_NBUF)]
            + [pltpu.SemaphoreType.DMA for _ in range(2 * _NBUF)]
        ),
    )
    def gather_kernel(idx_hbm, table_hbm, out_hbm, idx_v, *rest):
        bufs = rest[:_NBUF]
        gsems = rest[_NBUF:2 * _NBUF]
        ssems = rest[2 * _NBUF:3 * _NBUF]
        wid = lax.axis_index("s") * _NC + lax.axis_index("c")
        base = wid * b_per_w
        pltpu.sync_copy(idx_hbm.at[wid], idx_v)

        def gather(j):
            return pltpu.async_copy(
                table_hbm.at[idx_v.at[j]], bufs[j % _NBUF], gsems[j % _NBUF]
            )

        def scatter(j):
            return pltpu.async_copy(
                bufs[j % _NBUF],
                out_hbm.at[pl.ds(base + j * _CHUNK, _CHUNK)],
                ssems[j % _NBUF],
            )

        gathers = [None] * n_ch
        scatters = [None] * n_ch
        s_waited = [False] * n_ch
        prime = min(_INFLIGHT, n_ch)
        for j in range(prime):
            gathers[j] = gather(j)
        for j in range(n_ch):
            gathers[j].wait()
            scatters[j] = scatter(j)
            nx = j + prime
            if nx < n_ch:
                prev = nx - _NBUF
                if prev >= 0:
                    scatters[prev].wait()
                    s_waited[prev] = True
                gathers[nx] = gather(nx)
        for j in range(n_ch):
            if not s_waited[j]:
                scatters[j].wait()

    return gather_kernel


def kernel(x, W):
    Bx, S = x.shape
    D = W.shape[1]
    idx = x.reshape(-1).astype(jnp.int32)
    B = idx.size
    idx3 = idx.reshape(_NW, B // _NW // _CHUNK, _CHUNK)
    out = _make_gather(B, D)(idx3, W)
    return out.reshape(Bx, S, D)
